# baseline (speedup 1.0000x reference)
import functools

import jax
import jax.numpy as jnp
from jax.experimental import pallas as pl
from jax.experimental.pallas import tpu as pltpu
from jax.experimental.pallas import tpu_sc as plsc

_N = 10000
_E = 320000
_DE = 16
_DN = 128
_DG = 16

_NC = 2
_NS = 16
_L = 16
_NW = _NC * _NS

_CHUNK = 128
_NROWS = _E // _CHUNK
_RPW = 80
_NCHP = _RPW * _NW
_NPAD = _N + _L
_NPS = _NPAD // _NS
_EB_ROWS = _E // 8


def _pre_node_body(node_ref, wr_ref, ws_ref, pr_ref, ps_ref):
    x = node_ref[...]
    pr_ref[...] = jnp.dot(x, wr_ref[...], preferred_element_type=jnp.float32)
    ps_ref[...] = jnp.dot(x, ws_ref[...], preferred_element_type=jnp.float32)


def _pre_edge_body(e_ref, wk_ref, c_ref, o_ref):
    o_ref[...] = (
        jnp.dot(e_ref[...], wk_ref[...], preferred_element_type=jnp.float32)
        + c_ref[...]
    )


_W = 10
_NWIN = _RPW // _W


def _sc_edge_kernel(idx_hbm, pr_hbm, ps_hbm, eb_hbm, eout_hbm, agg_hbm,
                    idxw, prb, psb, ebw, outbw, outb2w, zbuf,
                    agg_sh, semgs, sem_eb, semo):
    sid = jax.lax.axis_index("s")
    cid = jax.lax.axis_index("c")
    wid = cid * _NS + sid
    c0 = wid * _RPW

    @pl.loop(0, _NPS)
    def _(i):
        zbuf[i, :] = jnp.zeros((_L,), jnp.float32)

    pltpu.sync_copy(zbuf, agg_sh.at[pl.ds(sid * _NPS, _NPS), :])
    plsc.subcore_barrier()

    @pl.loop(0, _NWIN)
    def _(w):
        q0 = c0 + w * _W
        qe = jnp.minimum(q0, _NROWS - _W) * 16

        pltpu.sync_copy(idx_hbm.at[pl.ds(q0 * 2, 2 * _W), :], idxw)
        gds = []
        for i in range(_W):
            gds.append((
                pltpu.async_copy(pr_hbm.at[idxw.at[2 * i + 1]],
                                 prb.at[pl.ds(i * _CHUNK, _CHUNK), :],
                                 semgs[i]),
                pltpu.async_copy(ps_hbm.at[idxw.at[2 * i]],
                                 psb.at[pl.ds(i * _CHUNK, _CHUNK), :],
                                 semgs[i]),
            ))
        ebd = pltpu.async_copy(eb_hbm.at[pl.ds(qe, 16 * _W), :], ebw, sem_eb)
        ebd.wait()

        sds = []
        for i in range(_W):
            gds[i][0].wait()
            gds[i][1].wait()

            @pl.loop(0, 16)
            def _(a):
                for j in range(8):
                    e = i * _CHUNK + a * 8 + j
                    v = prb[e, :] + psb[e, :] \
                        + ebw[i * 16 + a, pl.ds(j * 16, 16)]
                    v = jnp.maximum(v, 0.0)
                    outbw[e, :] = v
                    outb2w[i * 16 + a, pl.ds(j * 16, 16)] = v

            sds.append(pltpu.async_copy(
                outbw.at[pl.ds(i * _CHUNK, _CHUNK), :],
                agg_sh.at[idxw.at[2 * i + 1]], semo, add=True))

        @pl.when(q0 < _NROWS)
        def _():
            pltpu.sync_copy(outb2w, eout_hbm.at[pl.ds(qe, 16 * _W), :])

        for d in sds:
            d.wait()

    plsc.subcore_barrier()
    pltpu.sync_copy(agg_sh.at[pl.ds(sid * _NPS, _NPS), :],
                    agg_hbm.at[cid, pl.ds(sid * _NPS, _NPS), :])


def _conv_body(e_ref, o_ref):
    o_ref[...] = e_ref[...]


def _aggconv_body(a_ref, o_ref):
    o_ref[...] = a_ref[0, :, :] + a_ref[1, :, :]


def _post_body(node_ref, a_ref, wn_ref, wg_ref, cn_ref, g_ref,
               bg_ref, nout_ref, gout_ref, nsum, esum):
    i = pl.program_id(0)
    agg = a_ref[...]
    x = node_ref[...]
    h = jnp.dot(x, wn_ref[0:128, :], preferred_element_type=jnp.float32)
    h = h + jnp.dot(agg, wn_ref[128:144, :],
                    preferred_element_type=jnp.float32)
    nn = jnp.maximum(h + cn_ref[...], 0.0)
    nout_ref[...] = nn

    @pl.when(i == 0)
    def _():
        nsum[...] = jnp.zeros_like(nsum)
        esum[...] = jnp.zeros_like(esum)

    nsum[...] += jnp.sum(nn, axis=0, keepdims=True)
    esum[...] += jnp.sum(agg, axis=0, keepdims=True)

    @pl.when(i == pl.num_programs(0) - 1)
    def _():
        gi = jnp.dot(nsum[...], wg_ref[0:128, :],
                     preferred_element_type=jnp.float32)
        gi = gi + jnp.dot(esum[...], wg_ref[128:144, :],
                          preferred_element_type=jnp.float32)
        gi = gi + jnp.dot(g_ref[...], wg_ref[144:160, :],
                          preferred_element_type=jnp.float32)
        gout_ref[...] = jnp.maximum(gi + bg_ref[...], 0.0)


@jax.jit
def kernel(edge_attr, node_attr, global_attr, edge_index, batch,
           W_e, b_e, W_n, b_n, W_g, b_g):
    f32 = jnp.float32

    Wr = W_e[0:_DN, :]
    Ws = W_e[_DN:2 * _DN, :]
    Wee = W_e[2 * _DN:2 * _DN + _DE, :]
    ce = global_attr @ W_e[2 * _DN + _DE:, :] + b_e
    cvec = jnp.tile(ce, (1, 8))
    Wk = jnp.kron(jnp.eye(8, dtype=f32), Wee)
    cn = global_attr @ W_n[_DN + _DE:, :] + b_n

    edge2d = edge_attr.reshape(_EB_ROWS, 8 * _DE)
    idx2 = jnp.pad(
        edge_index.reshape(2, _NROWS, _CHUNK),
        ((0, 0), (0, _NCHP - _NROWS), (0, 0)),
        constant_values=_N,
    )
    idxp = jnp.stack([idx2[0], idx2[1]], axis=1).reshape(2 * _NCHP, _CHUNK)

    pr, ps = pl.pallas_call(
        _pre_node_body,
        grid=(10,),
        in_specs=[
            pl.BlockSpec((1000, _DN), lambda i: (i, 0)),
            pl.BlockSpec((_DN, _DE), lambda i: (0, 0)),
            pl.BlockSpec((_DN, _DE), lambda i: (0, 0)),
        ],
        out_specs=[
            pl.BlockSpec((1000, _DE), lambda i: (i, 0)),
            pl.BlockSpec((1000, _DE), lambda i: (i, 0)),
        ],
        out_shape=[jax.ShapeDtypeStruct((_NPAD, _DE), f32)] * 2,
    )(node_attr, Wr, Ws)

    ebase = pl.pallas_call(
        _pre_edge_body,
        grid=(40,),
        in_specs=[
            pl.BlockSpec((1000, 128), lambda i: (i, 0)),
            pl.BlockSpec((128, 128), lambda i: (0, 0)),
            pl.BlockSpec((1, 128), lambda i: (0, 0)),
        ],
        out_specs=pl.BlockSpec((1000, 128), lambda i: (i, 0)),
        out_shape=jax.ShapeDtypeStruct((_EB_ROWS, 8 * _DE), f32),
    )(edge2d, Wk, cvec)

    sc_edge = functools.partial(
        pl.kernel,
        out_type=(
            jax.ShapeDtypeStruct((_EB_ROWS, 8 * _DE), f32),
            jax.ShapeDtypeStruct((_NC, _NPAD, _DE), f32),
        ),
        mesh=plsc.VectorSubcoreMesh(core_axis_name="c", subcore_axis_name="s"),
        compiler_params=pltpu.CompilerParams(use_tc_tiling_on_sc=False),
        scratch_types=[
            pltpu.VMEM((2 * _W, _CHUNK), jnp.int32),
            pltpu.VMEM((_W * _CHUNK, _DE), f32),
            pltpu.VMEM((_W * _CHUNK, _DE), f32),
            pltpu.VMEM((16 * _W, 8 * _DE), f32),
            pltpu.VMEM((_W * _CHUNK, _DE), f32),
            pltpu.VMEM((16 * _W, 8 * _DE), f32),
            pltpu.VMEM((_NPS, _DE), f32),
            pltpu.VMEM_SHARED((_NPAD, _DE), f32),
            [pltpu.SemaphoreType.DMA] * _W,
            pltpu.SemaphoreType.DMA,
            pltpu.SemaphoreType.DMA,
        ],
    )(_sc_edge_kernel)
    eout128, agg2 = sc_edge(idxp, pr, ps, ebase)

    eout_t = pl.pallas_call(
        _conv_body,
        grid=(40,),
        in_specs=[pl.BlockSpec((1000, 128), lambda i: (i, 0))],
        out_specs=pl.BlockSpec((1000, 128), lambda i: (i, 0)),
        out_shape=jax.ShapeDtypeStruct((_EB_ROWS, 8 * _DE), f32),
    )(eout128)
    edge_new = eout_t.reshape(_E, _DE)

    agg128 = agg2.reshape(_NC, _NPAD * _DE // 128, 128)
    aggsum = pl.pallas_call(
        _aggconv_body,
        grid=(1,),
        in_specs=[pl.BlockSpec((_NC, _NPAD * _DE // 128, 128),
                               lambda i: (0, 0, 0))],
        out_specs=pl.BlockSpec((_NPAD * _DE // 128, 128), lambda i: (0, 0)),
        out_shape=jax.ShapeDtypeStruct((_NPAD * _DE // 128, 128), f32),
    )(agg128)
    agg_t = aggsum.reshape(_NPAD, _DE)

    node_new, global_new = pl.pallas_call(
        _post_body,
        grid=(10,),
        in_specs=[
            pl.BlockSpec((1000, _DN), lambda i: (i, 0)),
            pl.BlockSpec((1000, _DE), lambda i: (i, 0)),
            pl.BlockSpec((_DN + _DE + _DG, _DN), lambda i: (0, 0)),
            pl.BlockSpec((_DN + _DE + _DG, _DG), lambda i: (0, 0)),
            pl.BlockSpec((1, _DN), lambda i: (0, 0)),
            pl.BlockSpec((1, _DG), lambda i: (0, 0)),
            pl.BlockSpec((1, _DG), lambda i: (0, 0)),
        ],
        out_specs=[
            pl.BlockSpec((1000, _DN), lambda i: (i, 0)),
            pl.BlockSpec((1, _DG), lambda i: (0, 0)),
        ],
        out_shape=[
            jax.ShapeDtypeStruct((_N, _DN), f32),
            jax.ShapeDtypeStruct((1, _DG), f32),
        ],
        scratch_shapes=[
            pltpu.VMEM((1, _DN), f32),
            pltpu.VMEM((1, _DG), f32),
        ],
    )(node_attr, agg_t, W_n, W_g, cn, global_attr,
      b_g.reshape(1, _DG))

    return edge_new, node_new, global_new


# speedup vs baseline: 1.0607x; 1.0607x over previous
import functools

import jax
import jax.numpy as jnp
from jax.experimental import pallas as pl
from jax.experimental.pallas import tpu as pltpu
from jax.experimental.pallas import tpu_sc as plsc

_N = 10000
_E = 320000
_DE = 16
_DN = 128
_DG = 16

_NC = 2
_NS = 16
_L = 16
_NW = _NC * _NS

_CHUNK = 128
_NROWS = _E // _CHUNK
_RPW = 80
_NCHP = _RPW * _NW
_NPAD = _N + _L
_NPS = _NPAD // _NS
_EB_ROWS = _E // 8
_W = 10
_NWIN = _RPW // _W
_WE = _W * _CHUNK


def _pre_node_body(node_ref, wr_ref, ws_ref, pr_ref, ps_ref):
    x = node_ref[...]
    pr_ref[...] = jnp.dot(x, wr_ref[...], preferred_element_type=jnp.float32)
    ps_ref[...] = jnp.dot(x, ws_ref[...], preferred_element_type=jnp.float32)


def _pre_edge_body(e_ref, wk_ref, c_ref, o_ref):
    o_ref[...] = (
        jnp.dot(e_ref[...], wk_ref[...], preferred_element_type=jnp.float32)
        + c_ref[...]
    )


def _sc_edge_kernel(idx_hbm, pr_hbm, ps_hbm, eb_hbm, eout_hbm, agg_hbm,
                    idxw, prb, psb, ebw, outbw, zbuf,
                    agg_sh, semgs, sem_eb, sem_idx, sem_out, semo):
    sid = jax.lax.axis_index("s")
    cid = jax.lax.axis_index("c")
    wid = cid * _NS + sid
    c0 = wid * _RPW

    @pl.loop(0, _NPS)
    def _(i):
        zbuf[i, :] = jnp.zeros((_L,), jnp.float32)

    pltpu.sync_copy(zbuf, agg_sh.at[pl.ds(sid * _NPS, _NPS), :])
    plsc.subcore_barrier()

    def idx_copy(w, b):
        return pltpu.make_async_copy(
            idx_hbm.at[pl.ds((c0 + w * _W) * 2, 2 * _W), :], idxw[b],
            sem_idx[b])

    def out_copy(w, b):
        q0 = c0 + w * _W
        qe = jnp.minimum(q0, _NROWS - _W) * _CHUNK
        return pltpu.make_async_copy(
            outbw[b], eout_hbm.at[pl.ds(qe, _WE), :], sem_out[b])

    idx_copy(0, 0).start()

    @pl.loop(0, _NWIN)
    def _(w):
        b = jax.lax.rem(w, 2)
        q0 = c0 + w * _W
        qe = jnp.minimum(q0, _NROWS - _W) * 16

        for bb in (0, 1):
            @pl.when(b == bb)
            def _():
                idx_copy(w, bb).wait()

                @pl.when(w + 1 < _NWIN)
                def _():
                    idx_copy(w + 1, 1 - bb).start()

                gds = []
                for i in range(_W):
                    gds.append((
                        pltpu.async_copy(
                            pr_hbm.at[idxw[bb].at[2 * i + 1]],
                            prb.at[pl.ds(i * _CHUNK, _CHUNK), :], semgs[i]),
                        pltpu.async_copy(
                            ps_hbm.at[idxw[bb].at[2 * i]],
                            psb.at[pl.ds(i * _CHUNK, _CHUNK), :], semgs[i]),
                    ))
                ebd = pltpu.async_copy(eb_hbm.at[pl.ds(qe, 16 * _W), :],
                                       ebw, sem_eb)

                @pl.when((w >= 2) & (c0 + (w - 2) * _W < _NROWS))
                def _():
                    out_copy(w - 2, bb).wait()

                ebd.wait()
                sds = []
                for i in range(_W):
                    gds[i][0].wait()
                    gds[i][1].wait()

                    @pl.loop(0, 16)
                    def _(a):
                        for j in range(8):
                            e = i * _CHUNK + a * 8 + j
                            v = prb[e, :] + psb[e, :] \
                                + ebw[i * 16 + a, pl.ds(j * 16, 16)]
                            outbw[bb][e, :] = jnp.maximum(v, 0.0)

                    sds.append(pltpu.async_copy(
                        outbw[bb].at[pl.ds(i * _CHUNK, _CHUNK), :],
                        agg_sh.at[idxw[bb].at[2 * i + 1]], semo, add=True))

                @pl.when(q0 < _NROWS)
                def _():
                    out_copy(w, bb).start()

                for d in sds:
                    d.wait()

    @pl.when(c0 + (_NWIN - 2) * _W < _NROWS)
    def _():
        out_copy(_NWIN - 2, _NWIN % 2).wait()

    @pl.when(c0 + (_NWIN - 1) * _W < _NROWS)
    def _():
        out_copy(_NWIN - 1, (_NWIN - 1) % 2).wait()

    plsc.subcore_barrier()
    pltpu.sync_copy(agg_sh.at[pl.ds(sid * _NPS, _NPS), :],
                    agg_hbm.at[cid, pl.ds(sid * _NPS, _NPS), :])


def _post_body(node_ref, a0_ref, a1_ref, wn_ref, wg_ref, cn_ref, g_ref,
               bg_ref, nout_ref, gout_ref, nsum, esum):
    i = pl.program_id(0)
    agg = a0_ref[...] + a1_ref[...]
    x = node_ref[...]
    h = jnp.dot(x, wn_ref[0:128, :], preferred_element_type=jnp.float32)
    h = h + jnp.dot(agg, wn_ref[128:144, :],
                    preferred_element_type=jnp.float32)
    nn = jnp.maximum(h + cn_ref[...], 0.0)
    nout_ref[...] = nn

    @pl.when(i == 0)
    def _():
        nsum[...] = jnp.zeros_like(nsum)
        esum[...] = jnp.zeros_like(esum)

    nsum[...] += jnp.sum(nn, axis=0, keepdims=True)
    esum[...] += jnp.sum(agg, axis=0, keepdims=True)

    @pl.when(i == pl.num_programs(0) - 1)
    def _():
        gi = jnp.dot(nsum[...], wg_ref[0:128, :],
                     preferred_element_type=jnp.float32)
        gi = gi + jnp.dot(esum[...], wg_ref[128:144, :],
                          preferred_element_type=jnp.float32)
        gi = gi + jnp.dot(g_ref[...], wg_ref[144:160, :],
                          preferred_element_type=jnp.float32)
        gout_ref[...] = jnp.maximum(gi + bg_ref[...], 0.0)


@jax.jit
def kernel(edge_attr, node_attr, global_attr, edge_index, batch,
           W_e, b_e, W_n, b_n, W_g, b_g):
    f32 = jnp.float32

    Wr = W_e[0:_DN, :]
    Ws = W_e[_DN:2 * _DN, :]
    Wee = W_e[2 * _DN:2 * _DN + _DE, :]
    ce = global_attr @ W_e[2 * _DN + _DE:, :] + b_e
    cvec = jnp.tile(ce, (1, 8))
    Wk = jnp.kron(jnp.eye(8, dtype=f32), Wee)
    cn = global_attr @ W_n[_DN + _DE:, :] + b_n

    edge2d = edge_attr.reshape(_EB_ROWS, 8 * _DE)
    idx2 = jnp.pad(
        edge_index.reshape(2, _NROWS, _CHUNK),
        ((0, 0), (0, _NCHP - _NROWS), (0, 0)),
        constant_values=_N,
    )
    idxp = jnp.stack([idx2[0], idx2[1]], axis=1).reshape(2 * _NCHP, _CHUNK)

    pr, ps = pl.pallas_call(
        _pre_node_body,
        grid=(10,),
        in_specs=[
            pl.BlockSpec((1000, _DN), lambda i: (i, 0)),
            pl.BlockSpec((_DN, _DE), lambda i: (0, 0)),
            pl.BlockSpec((_DN, _DE), lambda i: (0, 0)),
        ],
        out_specs=[
            pl.BlockSpec((1000, _DE), lambda i: (i, 0)),
            pl.BlockSpec((1000, _DE), lambda i: (i, 0)),
        ],
        out_shape=[jax.ShapeDtypeStruct((_NPAD, _DE), f32)] * 2,
    )(node_attr, Wr, Ws)

    ebase = pl.pallas_call(
        _pre_edge_body,
        grid=(40,),
        in_specs=[
            pl.BlockSpec((1000, 128), lambda i: (i, 0)),
            pl.BlockSpec((128, 128), lambda i: (0, 0)),
            pl.BlockSpec((1, 128), lambda i: (0, 0)),
        ],
        out_specs=pl.BlockSpec((1000, 128), lambda i: (i, 0)),
        out_shape=jax.ShapeDtypeStruct((_EB_ROWS, 8 * _DE), f32),
    )(edge2d, Wk, cvec)

    sc_edge = functools.partial(
        pl.kernel,
        out_type=(
            jax.ShapeDtypeStruct((_E, _DE), f32),
            jax.ShapeDtypeStruct((_NC, _NPAD, _DE), f32),
        ),
        mesh=plsc.VectorSubcoreMesh(core_axis_name="c", subcore_axis_name="s"),
        compiler_params=pltpu.CompilerParams(use_tc_tiling_on_sc=False),
        scratch_types=[
            [pltpu.VMEM((2 * _W, _CHUNK), jnp.int32)] * 2,
            pltpu.VMEM((_WE, _DE), f32),
            pltpu.VMEM((_WE, _DE), f32),
            pltpu.VMEM((16 * _W, 8 * _DE), f32),
            [pltpu.VMEM((_WE, _DE), f32)] * 2,
            pltpu.VMEM((_NPS, _DE), f32),
            pltpu.VMEM_SHARED((_NPAD, _DE), f32),
            [pltpu.SemaphoreType.DMA] * _W,
            pltpu.SemaphoreType.DMA,
            [pltpu.SemaphoreType.DMA] * 2,
            [pltpu.SemaphoreType.DMA] * 2,
            pltpu.SemaphoreType.DMA,
        ],
    )(_sc_edge_kernel)
    edge_new, agg2 = sc_edge(idxp, pr, ps, ebase)

    node_new, global_new = pl.pallas_call(
        _post_body,
        grid=(10,),
        in_specs=[
            pl.BlockSpec((1000, _DN), lambda i: (i, 0)),
            pl.BlockSpec((1000, _DE), lambda i: (i, 0)),
            pl.BlockSpec((1000, _DE), lambda i: (i, 0)),
            pl.BlockSpec((_DN + _DE + _DG, _DN), lambda i: (0, 0)),
            pl.BlockSpec((_DN + _DE + _DG, _DG), lambda i: (0, 0)),
            pl.BlockSpec((1, _DN), lambda i: (0, 0)),
            pl.BlockSpec((1, _DG), lambda i: (0, 0)),
            pl.BlockSpec((1, _DG), lambda i: (0, 0)),
        ],
        out_specs=[
            pl.BlockSpec((1000, _DN), lambda i: (i, 0)),
            pl.BlockSpec((1, _DG), lambda i: (0, 0)),
        ],
        out_shape=[
            jax.ShapeDtypeStruct((_N, _DN), f32),
            jax.ShapeDtypeStruct((1, _DG), f32),
        ],
        scratch_shapes=[
            pltpu.VMEM((1, _DN), f32),
            pltpu.VMEM((1, _DG), f32),
        ],
    )(node_attr, agg2[0], agg2[1], W_n, W_g, cn, global_attr,
      b_g.reshape(1, _DG))

    return edge_new, node_new, global_new
